# baseline (speedup 1.0000x reference)
import functools

import jax
import jax.numpy as jnp
from jax import lax
from jax.experimental import pallas as pl
from jax.experimental.pallas import tpu as pltpu
from jax.experimental.pallas import tpu_sc as plsc

BATCH = 16384
D_EMB = 128
CHUNKS = (12288, 4096)

_NUM_CORES = 2
_NUM_SUBCORES = 16
_NUM_WORKERS = _NUM_CORES * _NUM_SUBCORES
_CHUNK = 128


@functools.cache
def _make_sc_gather(offset, size):
    bpw = size // _NUM_WORKERS
    nchunk = bpw // _CHUNK
    mesh = plsc.VectorSubcoreMesh(core_axis_name="c", subcore_axis_name="s")

    @functools.partial(
        pl.kernel,
        mesh=mesh,
        out_type=(
            jax.ShapeDtypeStruct((size, D_EMB), jnp.float32),
            jax.ShapeDtypeStruct((size, D_EMB), jnp.float32),
        ),
        scratch_types=[
            pltpu.VMEM((bpw,), jnp.int32),
            pltpu.VMEM((bpw,), jnp.int32),
            pltpu.VMEM((bpw, D_EMB), jnp.float32),
            pltpu.VMEM((bpw, D_EMB), jnp.float32),
            pltpu.SemaphoreType.DMA,
            pltpu.SemaphoreType.DMA,
        ],
    )
    def _sc_gather(users_hbm, movies_hbm, ut_hbm, mt_hbm, u_out, m_out,
                   uidx_v, midx_v, urows_v, mrows_v, usem, msem):
        wid = lax.axis_index("s") * _NUM_CORES + lax.axis_index("c")
        base = wid * bpw
        src_base = offset + wid * bpw
        pltpu.sync_copy(users_hbm.at[pl.ds(src_base, bpw)], uidx_v)
        pltpu.sync_copy(movies_hbm.at[pl.ds(src_base, bpw)], midx_v)

        def fire(table_hbm, idx_v, rows_v, sem):
            return [
                pltpu.async_copy(
                    table_hbm.at[idx_v.at[pl.ds(j * _CHUNK, _CHUNK)]],
                    rows_v.at[pl.ds(j * _CHUNK, _CHUNK)],
                    sem,
                )
                for j in range(nchunk)
            ]

        ucopies = fire(ut_hbm, uidx_v, urows_v, usem)
        mcopies = fire(mt_hbm, midx_v, mrows_v, msem)
        for cp in ucopies:
            cp.wait()
        pltpu.sync_copy(urows_v, u_out.at[pl.ds(base, bpw)])
        for cp in mcopies:
            cp.wait()
        pltpu.sync_copy(mrows_v, m_out.at[pl.ds(base, bpw)])

    return _sc_gather


_BB = 2048


def _mlp_body(u_ref, m_ref, w0_ref, b0_ref, w1_ref, b1_ref,
              wout_ref, bout_ref, o_ref):
    bf = jnp.bfloat16
    u = u_ref[...].astype(bf)
    m = m_ref[...].astype(bf)
    h0 = jnp.dot(u, w0_ref[0:D_EMB, :], preferred_element_type=jnp.float32)
    h0 += jnp.dot(m, w0_ref[D_EMB:2 * D_EMB, :],
                  preferred_element_type=jnp.float32)
    h0 = jnp.maximum(h0 + b0_ref[...], 0.0).astype(bf)
    h1 = jnp.dot(h0, w1_ref[...], preferred_element_type=jnp.float32)
    h1 = jnp.maximum(h1 + b1_ref[...], 0.0).astype(bf)
    out = jnp.dot(h1, wout_ref[...], preferred_element_type=jnp.float32)
    out = out + bout_ref[...]
    o_ref[...] = out.reshape(1, out.shape[0])


def _mlp(u_emb, m_emb, W0, b0, W1, b1, Wout, bout):
    size = u_emb.shape[0]
    h0_dim = W0.shape[1]
    h1_dim = W1.shape[1]
    grid = (size // _BB,)
    return pl.pallas_call(
        _mlp_body,
        grid=grid,
        in_specs=[
            pl.BlockSpec((_BB, D_EMB), lambda i: (i, 0)),
            pl.BlockSpec((_BB, D_EMB), lambda i: (i, 0)),
            pl.BlockSpec((2 * D_EMB, h0_dim), lambda i: (0, 0)),
            pl.BlockSpec((1, h0_dim), lambda i: (0, 0)),
            pl.BlockSpec((h0_dim, h1_dim), lambda i: (0, 0)),
            pl.BlockSpec((1, h1_dim), lambda i: (0, 0)),
            pl.BlockSpec((h1_dim, 1), lambda i: (0, 0)),
            pl.BlockSpec((1, 1), lambda i: (0, 0)),
        ],
        out_specs=pl.BlockSpec((1, _BB), lambda i: (0, i)),
        out_shape=jax.ShapeDtypeStruct((1, size), jnp.float32),
        compiler_params=pltpu.CompilerParams(
            dimension_semantics=("arbitrary",),
        ),
    )(u_emb, m_emb, W0, b0, W1, b1, Wout, bout)


def kernel(users, movies, user_table, movie_table, W0, b0, W1, b1, Wout, bout):
    users = users.astype(jnp.int32)
    movies = movies.astype(jnp.int32)
    W0 = W0.astype(jnp.bfloat16)
    W1 = W1.astype(jnp.bfloat16)
    Wout = Wout.astype(jnp.bfloat16)
    b0 = b0.reshape(1, -1)
    b1 = b1.reshape(1, -1)
    bout = bout.reshape(1, 1)
    outs = []
    offset = 0
    for size in CHUNKS:
        u_emb, m_emb = _make_sc_gather(offset, size)(
            users, movies, user_table, movie_table)
        outs.append(_mlp(u_emb, m_emb, W0, b0, W1, b1, Wout, bout))
        offset += size
    return jnp.concatenate(outs, axis=1).reshape(BATCH, 1)


# speedup vs baseline: 1.0439x; 1.0439x over previous
import functools

import jax
import jax.numpy as jnp
from jax import lax
from jax.experimental import pallas as pl
from jax.experimental.pallas import tpu as pltpu
from jax.experimental.pallas import tpu_sc as plsc

BATCH = 16384
D_EMB = 128
CHUNKS = (8192, 8192)

_NUM_CORES = 2
_NUM_SUBCORES = 16
_NUM_WORKERS = _NUM_CORES * _NUM_SUBCORES
_CHUNK = 128


@functools.cache
def _make_sc_gather(offset, size):
    bpw = size // _NUM_WORKERS
    nchunk = bpw // _CHUNK
    mesh = plsc.VectorSubcoreMesh(core_axis_name="c", subcore_axis_name="s")

    @functools.partial(
        pl.kernel,
        mesh=mesh,
        out_type=(
            jax.ShapeDtypeStruct((size, D_EMB), jnp.float32),
            jax.ShapeDtypeStruct((size, D_EMB), jnp.float32),
        ),
        scratch_types=[
            pltpu.VMEM((bpw,), jnp.int32),
            pltpu.VMEM((bpw,), jnp.int32),
            pltpu.VMEM((bpw, D_EMB), jnp.float32),
            pltpu.VMEM((bpw, D_EMB), jnp.float32),
            pltpu.SemaphoreType.DMA,
            pltpu.SemaphoreType.DMA,
        ],
    )
    def _sc_gather(users_hbm, movies_hbm, ut_hbm, mt_hbm, u_out, m_out,
                   uidx_v, midx_v, urows_v, mrows_v, usem, msem):
        wid = lax.axis_index("s") * _NUM_CORES + lax.axis_index("c")
        base = wid * bpw
        src_base = offset + wid * bpw
        pltpu.sync_copy(users_hbm.at[pl.ds(src_base, bpw)], uidx_v)
        pltpu.sync_copy(movies_hbm.at[pl.ds(src_base, bpw)], midx_v)

        def fire(table_hbm, idx_v, rows_v, sem):
            return [
                pltpu.async_copy(
                    table_hbm.at[idx_v.at[pl.ds(j * _CHUNK, _CHUNK)]],
                    rows_v.at[pl.ds(j * _CHUNK, _CHUNK)],
                    sem,
                )
                for j in range(nchunk)
            ]

        ucopies = fire(ut_hbm, uidx_v, urows_v, usem)
        mcopies = fire(mt_hbm, midx_v, mrows_v, msem)
        for cp in ucopies:
            cp.wait()
        pltpu.sync_copy(urows_v, u_out.at[pl.ds(base, bpw)])
        for cp in mcopies:
            cp.wait()
        pltpu.sync_copy(mrows_v, m_out.at[pl.ds(base, bpw)])

    return _sc_gather


_BB = 2048


def _mlp_body(u_ref, m_ref, w0_ref, b0_ref, w1_ref, b1_ref,
              wout_ref, bout_ref, o_ref):
    bf = jnp.bfloat16
    u = u_ref[...].astype(bf)
    m = m_ref[...].astype(bf)
    h0 = jnp.dot(u, w0_ref[0:D_EMB, :], preferred_element_type=jnp.float32)
    h0 += jnp.dot(m, w0_ref[D_EMB:2 * D_EMB, :],
                  preferred_element_type=jnp.float32)
    h0 = jnp.maximum(h0 + b0_ref[...], 0.0).astype(bf)
    h1 = jnp.dot(h0, w1_ref[...], preferred_element_type=jnp.float32)
    h1 = jnp.maximum(h1 + b1_ref[...], 0.0).astype(bf)
    out = jnp.dot(h1, wout_ref[...], preferred_element_type=jnp.float32)
    out = out + bout_ref[...]
    o_ref[...] = out.reshape(1, out.shape[0])


def _mlp(u_emb, m_emb, W0, b0, W1, b1, Wout, bout):
    size = u_emb.shape[0]
    h0_dim = W0.shape[1]
    h1_dim = W1.shape[1]
    grid = (size // _BB,)
    return pl.pallas_call(
        _mlp_body,
        grid=grid,
        in_specs=[
            pl.BlockSpec((_BB, D_EMB), lambda i: (i, 0)),
            pl.BlockSpec((_BB, D_EMB), lambda i: (i, 0)),
            pl.BlockSpec((2 * D_EMB, h0_dim), lambda i: (0, 0)),
            pl.BlockSpec((1, h0_dim), lambda i: (0, 0)),
            pl.BlockSpec((h0_dim, h1_dim), lambda i: (0, 0)),
            pl.BlockSpec((1, h1_dim), lambda i: (0, 0)),
            pl.BlockSpec((h1_dim, 1), lambda i: (0, 0)),
            pl.BlockSpec((1, 1), lambda i: (0, 0)),
        ],
        out_specs=pl.BlockSpec((1, _BB), lambda i: (0, i)),
        out_shape=jax.ShapeDtypeStruct((1, size), jnp.float32),
        compiler_params=pltpu.CompilerParams(
            dimension_semantics=("arbitrary",),
        ),
    )(u_emb, m_emb, W0, b0, W1, b1, Wout, bout)


def kernel(users, movies, user_table, movie_table, W0, b0, W1, b1, Wout, bout):
    users = users.astype(jnp.int32)
    movies = movies.astype(jnp.int32)
    W0 = W0.astype(jnp.bfloat16)
    W1 = W1.astype(jnp.bfloat16)
    Wout = Wout.astype(jnp.bfloat16)
    b0 = b0.reshape(1, -1)
    b1 = b1.reshape(1, -1)
    bout = bout.reshape(1, 1)
    outs = []
    offset = 0
    for size in CHUNKS:
        u_emb, m_emb = _make_sc_gather(offset, size)(
            users, movies, user_table, movie_table)
        outs.append(_mlp(u_emb, m_emb, W0, b0, W1, b1, Wout, bout))
        offset += size
    return jnp.concatenate(outs, axis=1).reshape(BATCH, 1)


# speedup vs baseline: 1.2517x; 1.1990x over previous
import functools

import jax
import jax.numpy as jnp
from jax import lax
from jax.experimental import pallas as pl
from jax.experimental.pallas import tpu as pltpu
from jax.experimental.pallas import tpu_sc as plsc

BATCH = 16384
D_EMB = 128
CHUNKS = (8192, 8192)

_NUM_CORES = 2
_NUM_SUBCORES = 16
_NUM_WORKERS = _NUM_CORES * _NUM_SUBCORES
_CHUNK = 128


@functools.cache
def _make_sc_gather(offset, size):
    bpw = size // _NUM_WORKERS
    nchunk = bpw // _CHUNK
    mesh = plsc.VectorSubcoreMesh(core_axis_name="c", subcore_axis_name="s")

    @functools.partial(
        pl.kernel,
        mesh=mesh,
        out_type=(
            jax.ShapeDtypeStruct((size, D_EMB), jnp.float32),
            jax.ShapeDtypeStruct((size, D_EMB), jnp.float32),
        ),
        scratch_types=[
            pltpu.VMEM((bpw,), jnp.int32),
            pltpu.VMEM((bpw,), jnp.int32),
            pltpu.VMEM((bpw, D_EMB), jnp.float32),
            pltpu.VMEM((bpw, D_EMB), jnp.float32),
            pltpu.SemaphoreType.DMA,
            pltpu.SemaphoreType.DMA,
        ],
    )
    def _sc_gather(users_hbm, movies_hbm, ut_hbm, mt_hbm, u_out, m_out,
                   uidx_v, midx_v, urows_v, mrows_v, usem, msem):
        wid = lax.axis_index("s") * _NUM_CORES + lax.axis_index("c")
        base = wid * bpw
        src_base = offset + wid * bpw
        pltpu.sync_copy(users_hbm.at[pl.ds(src_base, bpw)], uidx_v)
        pltpu.sync_copy(movies_hbm.at[pl.ds(src_base, bpw)], midx_v)

        def fire(table_hbm, idx_v, rows_v, sem):
            return [
                pltpu.async_copy(
                    table_hbm.at[idx_v.at[pl.ds(j * _CHUNK, _CHUNK)]],
                    rows_v.at[pl.ds(j * _CHUNK, _CHUNK)],
                    sem,
                )
                for j in range(nchunk)
            ]

        ucopies = fire(ut_hbm, uidx_v, urows_v, usem)
        mcopies = fire(mt_hbm, midx_v, mrows_v, msem)
        for cp in ucopies:
            cp.wait()
        pltpu.sync_copy(urows_v, u_out.at[pl.ds(base, bpw)])
        for cp in mcopies:
            cp.wait()
        pltpu.sync_copy(mrows_v, m_out.at[pl.ds(base, bpw)])

    return _sc_gather


_BB = 2048


def _mlp_body(u_ref, m_ref, w0_ref, b0_ref, w1_ref, b1_ref,
              wout_ref, bout_ref, o_ref):
    bf = jnp.bfloat16
    x = jnp.concatenate([u_ref[...], m_ref[...]], axis=1).astype(bf)
    h0 = jnp.dot(x, w0_ref[...], preferred_element_type=jnp.float32)
    h0 = jnp.maximum(h0 + b0_ref[...], 0.0).astype(bf)
    h1 = jnp.dot(h0, w1_ref[...], preferred_element_type=jnp.float32)
    h1 = jnp.maximum(h1 + b1_ref[...], 0.0).astype(bf)
    out = lax.dot_general(wout_ref[...], h1,
                          (((1,), (1,)), ((), ())),
                          preferred_element_type=jnp.float32)
    o_ref[...] = out + bout_ref[...]


def _mlp(u_emb, m_emb, W0, b0, W1, b1, Wout, bout):
    size = u_emb.shape[0]
    h0_dim = W0.shape[1]
    h1_dim = W1.shape[1]
    grid = (size // _BB,)
    return pl.pallas_call(
        _mlp_body,
        grid=grid,
        in_specs=[
            pl.BlockSpec((_BB, D_EMB), lambda i: (i, 0)),
            pl.BlockSpec((_BB, D_EMB), lambda i: (i, 0)),
            pl.BlockSpec((2 * D_EMB, h0_dim), lambda i: (0, 0)),
            pl.BlockSpec((1, h0_dim), lambda i: (0, 0)),
            pl.BlockSpec((h0_dim, h1_dim), lambda i: (0, 0)),
            pl.BlockSpec((1, h1_dim), lambda i: (0, 0)),
            pl.BlockSpec((1, h1_dim), lambda i: (0, 0)),
            pl.BlockSpec((1, 1), lambda i: (0, 0)),
        ],
        out_specs=pl.BlockSpec((1, _BB), lambda i: (0, i)),
        out_shape=jax.ShapeDtypeStruct((1, size), jnp.float32),
        compiler_params=pltpu.CompilerParams(
            dimension_semantics=("arbitrary",),
        ),
    )(u_emb, m_emb, W0, b0, W1, b1, Wout, bout)


def kernel(users, movies, user_table, movie_table, W0, b0, W1, b1, Wout, bout):
    users = users.astype(jnp.int32)
    movies = movies.astype(jnp.int32)
    W0 = W0.astype(jnp.bfloat16)
    W1 = W1.astype(jnp.bfloat16)
    Wout = Wout.astype(jnp.bfloat16).reshape(1, -1)
    b0 = b0.reshape(1, -1)
    b1 = b1.reshape(1, -1)
    bout = bout.reshape(1, 1)
    outs = []
    offset = 0
    for size in CHUNKS:
        u_emb, m_emb = _make_sc_gather(offset, size)(
            users, movies, user_table, movie_table)
        outs.append(_mlp(u_emb, m_emb, W0, b0, W1, b1, Wout, bout))
        offset += size
    return jnp.concatenate(outs, axis=1).reshape(BATCH, 1)
